# Initial kernel scaffold; baseline (speedup 1.0000x reference)
import jax
import jax.numpy as jnp
from jax.experimental import pallas as pl


def kernel(pos):
    raise NotImplementedError("write your pallas kernel here")


# speedup vs baseline: 83.7116x; 83.7116x over previous
import functools

import jax
import jax.numpy as jnp
from jax import lax
from jax.experimental import pallas as pl
from jax.experimental.pallas import tpu as pltpu
from jax.experimental.pallas import tpu_sc as plsc

_R = 256
_BIG = 1.0e30
_PADC = 1.0e4
_NW = 32
_L = 16


def _knn_body(npad, keys_ref, sqk_ref, q_ref, out_ref):
    i = pl.program_id(0)

    sqk = sqk_ref[0:1, :]
    q = q_ref[...]
    sqq = q[:, 3:4]
    dot = jax.lax.dot_general(q, keys_ref[...],
                              (((1,), (0,)), ((), ())),
                              preferred_element_type=jnp.float32)
    dist = (sqq + sqk) - 2.0 * dot

    col = lax.broadcasted_iota(jnp.int32, (_R, npad), 1)
    row = i * _R + lax.broadcasted_iota(jnp.int32, (_R, 1), 0)
    dist = jnp.where(col == row, _BIG, dist)

    for k in range(3):
        m = jnp.min(dist, axis=1, keepdims=True)
        cand = jnp.where(dist <= m, col, npad)
        idx = jnp.min(cand, axis=1, keepdims=True)
        out_ref[:, k:k + 1] = idx
        if k < 2:
            dist = jnp.where(col == idx, _BIG, dist)


def _rsqrt(s):
    s = jnp.maximum(s, 1e-35)
    h = 0.5 * s
    i = plsc.bitcast(s, jnp.int32)
    i = 0x5F3759DF - (i >> 1)
    y = plsc.bitcast(i, jnp.float32)
    for _ in range(3):
        y = y * (1.5 - h * y * y)
    return y


def _sc_body(npad, chunk, px_hbm, py_hbm, pz_hbm, c0_hbm, c1_hbm, c2_hbm,
             *refs):
    o_hbm = refs[0:9]
    px_v, py_v, pz_v, c0_v, c1_v, c2_v = refs[9:15]
    o_v = refs[15:24]
    wid = lax.axis_index("c") * (_NW // 2) + lax.axis_index("s")
    base = wid * chunk
    pltpu.sync_copy(px_hbm, px_v)
    pltpu.sync_copy(py_hbm, py_v)
    pltpu.sync_copy(pz_hbm, pz_v)
    pltpu.sync_copy(c0_hbm.at[pl.ds(base, chunk)], c0_v)
    pltpu.sync_copy(c1_hbm.at[pl.ds(base, chunk)], c1_v)
    pltpu.sync_copy(c2_hbm.at[pl.ds(base, chunk)], c2_v)

    eps = 1e-12

    def step(j, carry):
        off = j * _L
        i0 = c0_v[pl.ds(off, _L)]
        i1 = c1_v[pl.ds(off, _L)]
        i2 = c2_v[pl.ds(off, _L)]
        qx = px_v[pl.ds(base + off, _L)]
        qy = py_v[pl.ds(base + off, _L)]
        qz = pz_v[pl.ds(base + off, _L)]

        xx = plsc.load_gather(px_v, [i0]) - qx
        xy = plsc.load_gather(py_v, [i0]) - qy
        xz = plsc.load_gather(pz_v, [i0]) - qz
        yx = plsc.load_gather(px_v, [i1]) - qx
        yy = plsc.load_gather(py_v, [i1]) - qy
        yz = plsc.load_gather(pz_v, [i1]) - qz
        zx = plsc.load_gather(px_v, [i2]) - qx
        zy = plsc.load_gather(py_v, [i2]) - qy
        zz = plsc.load_gather(pz_v, [i2]) - qz

        s1 = xx * xx + xy * xy + xz * xz
        n1 = s1 * _rsqrt(s1) + eps
        e1x, e1y, e1z = xx / n1, xy / n1, xz / n1

        yd = yx * e1x + yy * e1y + yz * e1z
        ux, uy, uz = yx - yd * e1x, yy - yd * e1y, yz - yd * e1z
        s2 = ux * ux + uy * uy + uz * uz
        n2 = s2 * _rsqrt(s2) + eps
        e2x, e2y, e2z = ux / n2, uy / n2, uz / n2

        zd1 = zx * e1x + zy * e1y + zz * e1z
        zd2 = zx * e2x + zy * e2y + zz * e2z
        vx = zx - zd1 * e1x - zd2 * e2x
        vy = zy - zd1 * e1y - zd2 * e2y
        vz = zz - zd1 * e1z - zd2 * e2z
        s3 = vx * vx + vy * vy + vz * vz
        n3 = s3 * _rsqrt(s3) + eps
        e3x, e3y, e3z = vx / n3, vy / n3, vz / n3

        for k, v in enumerate((e1x, e1y, e1z, e2x, e2y, e2z,
                               e3x, e3y, e3z)):
            o_v[k][pl.ds(off, _L)] = v
        return carry

    lax.fori_loop(0, chunk // _L, step, 0)

    for k in range(9):
        pltpu.sync_copy(o_v[k], o_hbm[k].at[pl.ds(base, chunk)])


@jax.jit
def kernel(pos):
    n = pos.shape[0]
    align = max(_R, _NW * _L)
    npad = -(-n // align) * align

    sq = jnp.sum(pos * pos, axis=1)

    keys = jnp.full((128, npad), 0.0, jnp.float32)
    keys = keys.at[0:3, :n].set(pos.T)
    keys = keys.at[0:3, n:].set(_PADC)

    sqk = jnp.full((8, npad), 0.0, jnp.float32)
    sqk = sqk.at[0, :n].set(sq)
    sqk = sqk.at[0, n:].set(3.0e8)

    prow = jnp.zeros((npad, 128), jnp.float32)
    prow = prow.at[:n, 0:3].set(pos)
    prow = prow.at[:n, 3].set(sq)

    idx = pl.pallas_call(
        functools.partial(_knn_body, npad),
        grid=(npad // _R,),
        in_specs=[
            pl.BlockSpec((128, npad), lambda i: (0, 0)),
            pl.BlockSpec((8, npad), lambda i: (0, 0)),
            pl.BlockSpec((_R, 128), lambda i: (i, 0)),
        ],
        out_specs=pl.BlockSpec((_R, 128), lambda i: (i, 0)),
        out_shape=jax.ShapeDtypeStruct((npad, 128), jnp.int32),
    )(keys, sqk, prow)

    chunk = npad // _NW
    mesh = plsc.VectorSubcoreMesh(core_axis_name="c", subcore_axis_name="s")
    sc = functools.partial(
        pl.kernel,
        mesh=mesh,
        compiler_params=pltpu.CompilerParams(needs_layout_passes=False),
        out_type=[jax.ShapeDtypeStruct((npad,), jnp.float32)] * 9,
        scratch_types=[
            pltpu.VMEM((npad,), jnp.float32),
            pltpu.VMEM((npad,), jnp.float32),
            pltpu.VMEM((npad,), jnp.float32),
            pltpu.VMEM((chunk,), jnp.int32),
            pltpu.VMEM((chunk,), jnp.int32),
            pltpu.VMEM((chunk,), jnp.int32),
        ] + [pltpu.VMEM((chunk,), jnp.float32)] * 9,
    )(functools.partial(_sc_body, npad, chunk))

    outs = sc(prow[:, 0], prow[:, 1], prow[:, 2],
              idx[:, 0], idx[:, 1], idx[:, 2])

    return jnp.stack(outs, axis=1)[:n].reshape(n, 3, 3)


# speedup vs baseline: 92.4208x; 1.1040x over previous
import functools

import jax
import jax.numpy as jnp
from jax import lax
from jax.experimental import pallas as pl
from jax.experimental.pallas import tpu as pltpu
from jax.experimental.pallas import tpu_sc as plsc

_R = 256
_BIG = 1.0e30
_PADC = 1.0e4
_NW = 32
_L = 16


def _knn_body(npad, keys_ref, q_ref, out_ref):
    i = pl.program_id(0)

    sqk = keys_ref[4:5, :]
    q = q_ref[...]
    sqq = q[:, 3:4]
    dot = jax.lax.dot_general(q, keys_ref[...],
                              (((1,), (0,)), ((), ())),
                              preferred_element_type=jnp.float32)
    dist = (sqq + sqk) - 2.0 * dot

    col = lax.broadcasted_iota(jnp.int32, (_R, npad), 1)
    row = i * _R + lax.broadcasted_iota(jnp.int32, (_R, 1), 0)
    dist = jnp.where(col == row, _BIG, dist)

    for k in range(3):
        idx = jnp.argmin(dist, axis=1).astype(jnp.int32)[:, None]
        out_ref[:, k:k + 1] = idx
        if k < 2:
            dist = jnp.where(col == idx, _BIG, dist)


def _rsqrt(s):
    s = jnp.maximum(s, 1e-35)
    h = 0.5 * s
    i = plsc.bitcast(s, jnp.int32)
    i = 0x5F3759DF - (i >> 1)
    y = plsc.bitcast(i, jnp.float32)
    for _ in range(3):
        y = y * (1.5 - h * y * y)
    return y


def _sc_body(npad, chunk, px_hbm, py_hbm, pz_hbm, c0_hbm, c1_hbm, c2_hbm,
             *refs):
    o_hbm = refs[0:9]
    px_v, py_v, pz_v, c0_v, c1_v, c2_v = refs[9:15]
    o_v = refs[15:24]
    wid = lax.axis_index("c") * (_NW // 2) + lax.axis_index("s")
    base = wid * chunk
    pltpu.sync_copy(px_hbm, px_v)
    pltpu.sync_copy(py_hbm, py_v)
    pltpu.sync_copy(pz_hbm, pz_v)
    pltpu.sync_copy(c0_hbm.at[pl.ds(base, chunk)], c0_v)
    pltpu.sync_copy(c1_hbm.at[pl.ds(base, chunk)], c1_v)
    pltpu.sync_copy(c2_hbm.at[pl.ds(base, chunk)], c2_v)

    eps = 1e-12

    def step(j, carry):
        off = j * _L
        i0 = c0_v[pl.ds(off, _L)]
        i1 = c1_v[pl.ds(off, _L)]
        i2 = c2_v[pl.ds(off, _L)]
        qx = px_v[pl.ds(base + off, _L)]
        qy = py_v[pl.ds(base + off, _L)]
        qz = pz_v[pl.ds(base + off, _L)]

        xx = plsc.load_gather(px_v, [i0]) - qx
        xy = plsc.load_gather(py_v, [i0]) - qy
        xz = plsc.load_gather(pz_v, [i0]) - qz
        yx = plsc.load_gather(px_v, [i1]) - qx
        yy = plsc.load_gather(py_v, [i1]) - qy
        yz = plsc.load_gather(pz_v, [i1]) - qz
        zx = plsc.load_gather(px_v, [i2]) - qx
        zy = plsc.load_gather(py_v, [i2]) - qy
        zz = plsc.load_gather(pz_v, [i2]) - qz

        s1 = xx * xx + xy * xy + xz * xz
        n1 = s1 * _rsqrt(s1) + eps
        e1x, e1y, e1z = xx / n1, xy / n1, xz / n1

        yd = yx * e1x + yy * e1y + yz * e1z
        ux, uy, uz = yx - yd * e1x, yy - yd * e1y, yz - yd * e1z
        s2 = ux * ux + uy * uy + uz * uz
        n2 = s2 * _rsqrt(s2) + eps
        e2x, e2y, e2z = ux / n2, uy / n2, uz / n2

        zd1 = zx * e1x + zy * e1y + zz * e1z
        zd2 = zx * e2x + zy * e2y + zz * e2z
        vx = zx - zd1 * e1x - zd2 * e2x
        vy = zy - zd1 * e1y - zd2 * e2y
        vz = zz - zd1 * e1z - zd2 * e2z
        s3 = vx * vx + vy * vy + vz * vz
        n3 = s3 * _rsqrt(s3) + eps
        e3x, e3y, e3z = vx / n3, vy / n3, vz / n3

        for k, v in enumerate((e1x, e1y, e1z, e2x, e2y, e2z,
                               e3x, e3y, e3z)):
            o_v[k][pl.ds(off, _L)] = v
        return carry

    lax.fori_loop(0, chunk // _L, step, 0)

    for k in range(9):
        pltpu.sync_copy(o_v[k], o_hbm[k].at[pl.ds(base, chunk)])


@jax.jit
def kernel(pos):
    n = pos.shape[0]
    align = max(_R, _NW * _L)
    npad = -(-n // align) * align

    sq = jnp.sum(pos * pos, axis=1)

    keys = jnp.zeros((8, npad), jnp.float32)
    keys = keys.at[0:3, :n].set(pos.T)
    keys = keys.at[0:3, n:].set(_PADC)
    keys = keys.at[4, :n].set(sq)
    keys = keys.at[4, n:].set(3.0e8)

    prow = jnp.zeros((npad, 8), jnp.float32)
    prow = prow.at[:n, 0:3].set(pos)
    prow = prow.at[:n, 3].set(sq)

    idx = pl.pallas_call(
        functools.partial(_knn_body, npad),
        grid=(npad // _R,),
        in_specs=[
            pl.BlockSpec((8, npad), lambda i: (0, 0)),
            pl.BlockSpec((_R, 8), lambda i: (i, 0)),
        ],
        out_specs=pl.BlockSpec((_R, 8), lambda i: (i, 0)),
        out_shape=jax.ShapeDtypeStruct((npad, 8), jnp.int32),
    )(keys, prow)

    chunk = npad // _NW
    mesh = plsc.VectorSubcoreMesh(core_axis_name="c", subcore_axis_name="s")
    sc = functools.partial(
        pl.kernel,
        mesh=mesh,
        compiler_params=pltpu.CompilerParams(needs_layout_passes=False),
        out_type=[jax.ShapeDtypeStruct((npad,), jnp.float32)] * 9,
        scratch_types=[
            pltpu.VMEM((npad,), jnp.float32),
            pltpu.VMEM((npad,), jnp.float32),
            pltpu.VMEM((npad,), jnp.float32),
            pltpu.VMEM((chunk,), jnp.int32),
            pltpu.VMEM((chunk,), jnp.int32),
            pltpu.VMEM((chunk,), jnp.int32),
        ] + [pltpu.VMEM((chunk,), jnp.float32)] * 9,
    )(functools.partial(_sc_body, npad, chunk))

    outs = sc(prow[:, 0], prow[:, 1], prow[:, 2],
              idx[:, 0], idx[:, 1], idx[:, 2])

    return jnp.stack(outs, axis=1)[:n].reshape(n, 3, 3)


# speedup vs baseline: 96.2546x; 1.0415x over previous
import functools

import jax
import jax.numpy as jnp
from jax import lax
from jax.experimental import pallas as pl
from jax.experimental.pallas import tpu as pltpu
from jax.experimental.pallas import tpu_sc as plsc

_R = 256
_BIG = 1.0e30
_PADC = 1.0e4
_NW = 32
_L = 16


def _knn_body(npad, keys_ref, q_ref, out_ref):
    i = pl.program_id(0)

    sqk = keys_ref[4:5, :]
    q = q_ref[...]
    sqq = q[:, 3:4]
    dot = jax.lax.dot_general(q, keys_ref[...],
                              (((1,), (0,)), ((), ())),
                              preferred_element_type=jnp.float32)
    dist = (sqq + sqk) - 2.0 * dot

    colf = lax.broadcasted_iota(jnp.int32, (_R, npad), 1).astype(jnp.float32)
    rowf = (i * _R + lax.broadcasted_iota(jnp.int32, (_R, 1), 0)
            ).astype(jnp.float32)
    dist = jnp.where(colf == rowf, _BIG, dist)

    for k in range(3):
        m = jnp.min(dist, axis=1, keepdims=True)
        idxf = jnp.min(jnp.where(dist <= m, colf, 1.0e9),
                       axis=1, keepdims=True)
        out_ref[:, k:k + 1] = idxf.astype(jnp.int32)
        if k < 2:
            dist = jnp.where(colf == idxf, _BIG, dist)


def _rsqrt(s):
    s = jnp.maximum(s, 1e-35)
    h = 0.5 * s
    i = plsc.bitcast(s, jnp.int32)
    i = 0x5F3759DF - (i >> 1)
    y = plsc.bitcast(i, jnp.float32)
    for _ in range(3):
        y = y * (1.5 - h * y * y)
    return y


def _sc_body(npad, chunk, px_hbm, py_hbm, pz_hbm, c0_hbm, c1_hbm, c2_hbm,
             *refs):
    o_hbm = refs[0:9]
    px_v, py_v, pz_v, c0_v, c1_v, c2_v = refs[9:15]
    o_v = refs[15:24]
    wid = lax.axis_index("c") * (_NW // 2) + lax.axis_index("s")
    base = wid * chunk
    pltpu.sync_copy(px_hbm, px_v)
    pltpu.sync_copy(py_hbm, py_v)
    pltpu.sync_copy(pz_hbm, pz_v)
    pltpu.sync_copy(c0_hbm.at[pl.ds(base, chunk)], c0_v)
    pltpu.sync_copy(c1_hbm.at[pl.ds(base, chunk)], c1_v)
    pltpu.sync_copy(c2_hbm.at[pl.ds(base, chunk)], c2_v)

    eps = 1e-12

    def step(j, carry):
        off = j * _L
        i0 = c0_v[pl.ds(off, _L)]
        i1 = c1_v[pl.ds(off, _L)]
        i2 = c2_v[pl.ds(off, _L)]
        qx = px_v[pl.ds(base + off, _L)]
        qy = py_v[pl.ds(base + off, _L)]
        qz = pz_v[pl.ds(base + off, _L)]

        xx = plsc.load_gather(px_v, [i0]) - qx
        xy = plsc.load_gather(py_v, [i0]) - qy
        xz = plsc.load_gather(pz_v, [i0]) - qz
        yx = plsc.load_gather(px_v, [i1]) - qx
        yy = plsc.load_gather(py_v, [i1]) - qy
        yz = plsc.load_gather(pz_v, [i1]) - qz
        zx = plsc.load_gather(px_v, [i2]) - qx
        zy = plsc.load_gather(py_v, [i2]) - qy
        zz = plsc.load_gather(pz_v, [i2]) - qz

        s1 = xx * xx + xy * xy + xz * xz
        n1 = s1 * _rsqrt(s1) + eps
        e1x, e1y, e1z = xx / n1, xy / n1, xz / n1

        yd = yx * e1x + yy * e1y + yz * e1z
        ux, uy, uz = yx - yd * e1x, yy - yd * e1y, yz - yd * e1z
        s2 = ux * ux + uy * uy + uz * uz
        n2 = s2 * _rsqrt(s2) + eps
        e2x, e2y, e2z = ux / n2, uy / n2, uz / n2

        zd1 = zx * e1x + zy * e1y + zz * e1z
        zd2 = zx * e2x + zy * e2y + zz * e2z
        vx = zx - zd1 * e1x - zd2 * e2x
        vy = zy - zd1 * e1y - zd2 * e2y
        vz = zz - zd1 * e1z - zd2 * e2z
        s3 = vx * vx + vy * vy + vz * vz
        n3 = s3 * _rsqrt(s3) + eps
        e3x, e3y, e3z = vx / n3, vy / n3, vz / n3

        for k, v in enumerate((e1x, e1y, e1z, e2x, e2y, e2z,
                               e3x, e3y, e3z)):
            o_v[k][pl.ds(off, _L)] = v
        return carry

    lax.fori_loop(0, chunk // _L, step, 0)

    for k in range(9):
        pltpu.sync_copy(o_v[k], o_hbm[k].at[pl.ds(base, chunk)])


@jax.jit
def kernel(pos):
    n = pos.shape[0]
    align = max(_R, _NW * _L)
    npad = -(-n // align) * align

    sq = jnp.sum(pos * pos, axis=1)

    keys = jnp.zeros((8, npad), jnp.float32)
    keys = keys.at[0:3, :n].set(pos.T)
    keys = keys.at[0:3, n:].set(_PADC)
    keys = keys.at[4, :n].set(sq)
    keys = keys.at[4, n:].set(3.0e8)

    prow = jnp.zeros((npad, 8), jnp.float32)
    prow = prow.at[:n, 0:3].set(pos)
    prow = prow.at[:n, 3].set(sq)

    idx = pl.pallas_call(
        functools.partial(_knn_body, npad),
        grid=(npad // _R,),
        in_specs=[
            pl.BlockSpec((8, npad), lambda i: (0, 0)),
            pl.BlockSpec((_R, 8), lambda i: (i, 0)),
        ],
        out_specs=pl.BlockSpec((_R, 8), lambda i: (i, 0)),
        out_shape=jax.ShapeDtypeStruct((npad, 8), jnp.int32),
    )(keys, prow)

    chunk = npad // _NW
    mesh = plsc.VectorSubcoreMesh(core_axis_name="c", subcore_axis_name="s")
    sc = functools.partial(
        pl.kernel,
        mesh=mesh,
        compiler_params=pltpu.CompilerParams(needs_layout_passes=False),
        out_type=[jax.ShapeDtypeStruct((npad,), jnp.float32)] * 9,
        scratch_types=[
            pltpu.VMEM((npad,), jnp.float32),
            pltpu.VMEM((npad,), jnp.float32),
            pltpu.VMEM((npad,), jnp.float32),
            pltpu.VMEM((chunk,), jnp.int32),
            pltpu.VMEM((chunk,), jnp.int32),
            pltpu.VMEM((chunk,), jnp.int32),
        ] + [pltpu.VMEM((chunk,), jnp.float32)] * 9,
    )(functools.partial(_sc_body, npad, chunk))

    outs = sc(prow[:, 0], prow[:, 1], prow[:, 2],
              idx[:, 0], idx[:, 1], idx[:, 2])

    return jnp.stack(outs, axis=1)[:n].reshape(n, 3, 3)
